# baseline (speedup 1.0000x reference)
import functools

import jax
import jax.numpy as jnp
from jax import lax
from jax.experimental import pallas as pl
from jax.experimental.pallas import tpu as pltpu
from jax.experimental.pallas import tpu_sc as plsc

N = 10000
D = 128
H = 128
O = 64
E = 320000

NC = 2
NS = 16
NW = NC * NS
CH = 128
NCH = 79
EPW = NCH * CH
E_PAD = NW * EPW
N_PAD = 10240
RPT = N_PAD // NS
NR = 10112
RPA = NR // NS

_mesh = plsc.VectorSubcoreMesh(core_axis_name="c", subcore_axis_name="s")


def _fill_rows(ref, nrows, value):
    vec = jnp.full((16,), value, jnp.float32)

    def body(r, _):
        for cidx in range(CH // 16):
            ref[r, pl.ds(cidx * 16, 16)] = vec
        return 0

    lax.fori_loop(0, nrows, body, 0)


@functools.partial(
    pl.kernel,
    out_type=jax.ShapeDtypeStruct((NC, N_PAD), jnp.float32),
    mesh=_mesh,
    scratch_types=[
        pltpu.VMEM((NCH, CH), jnp.int32),
        pltpu.VMEM((CH,), jnp.float32),
        pltpu.VMEM((CH,), jnp.float32),
        pltpu.VMEM_SHARED((N_PAD,), jnp.float32),
    ],
)
def _sc_degree(dst_hbm, out_hbm, idx_v, ones_v, zero_v, deg_sh):
    cid = lax.axis_index("c")
    sid = lax.axis_index("s")
    wid = cid * NS + sid

    ones16 = jnp.ones((16,), jnp.float32)
    zeros16 = jnp.zeros((16,), jnp.float32)
    for i in range(CH // 16):
        ones_v[pl.ds(i * 16, 16)] = ones16
        zero_v[pl.ds(i * 16, 16)] = zeros16

    for k in range(RPT // CH):
        pltpu.sync_copy(zero_v, deg_sh.at[pl.ds(sid * RPT + k * CH, CH)])
    plsc.subcore_barrier()

    pltpu.sync_copy(dst_hbm.at[wid], idx_v)

    def body(j, _):
        pltpu.sync_copy(ones_v, deg_sh.at[idx_v.at[j]], add=True)
        return 0

    lax.fori_loop(0, NCH, body, 0)
    plsc.subcore_barrier()

    pltpu.sync_copy(
        deg_sh.at[pl.ds(sid * RPT, RPT)], out_hbm.at[cid, pl.ds(sid * RPT, RPT)]
    )


@functools.partial(
    pl.kernel,
    out_type=jax.ShapeDtypeStruct((NC, NR, H), jnp.float32),
    mesh=_mesh,
    scratch_types=[
        pltpu.VMEM((NCH, CH), jnp.int32),
        pltpu.VMEM((NCH, CH), jnp.int32),
        pltpu.VMEM((CH, H), jnp.float32),
        pltpu.VMEM_SHARED((NR, H), jnp.float32),
        pltpu.SemaphoreType.DMA,
    ],
)
def _sc_agg(hp_hbm, src_hbm, dst_hbm, out_hbm, src_v, dst_v, ra, agg_sh,
            sem_a):
    cid = lax.axis_index("c")
    sid = lax.axis_index("s")
    wid = cid * NS + sid

    _fill_rows(ra, CH, 0.0)
    base = sid * RPA
    for k in range(RPA // CH):
        pltpu.sync_copy(ra, agg_sh.at[pl.ds(base + k * CH, CH)])
    if RPA % CH:
        pltpu.sync_copy(ra, agg_sh.at[pl.ds(base + RPA - CH, CH)])
    plsc.subcore_barrier()

    pltpu.sync_copy(src_hbm.at[wid], src_v)
    pltpu.sync_copy(dst_hbm.at[wid], dst_v)

    def body(j, _):
        pltpu.async_copy(hp_hbm.at[src_v.at[j]], ra, sem_a)
        pltpu.make_async_copy(hp_hbm.at[src_v.at[j]], ra, sem_a).wait()
        pltpu.sync_copy(ra, agg_sh.at[dst_v.at[j]], add=True)
        return 0

    lax.fori_loop(0, NCH, body, 0)
    plsc.subcore_barrier()

    pltpu.sync_copy(
        agg_sh.at[pl.ds(sid * RPA, RPA)],
        out_hbm.at[cid, pl.ds(sid * RPA, RPA), :],
    )


_BLK = 2000
_GRID = N // _BLK


def _tc_dis(deg2):
    def body(deg_ref, o_ref):
        d = deg_ref[0, :] + deg_ref[1, :] + 1.0
        o_ref[...] = lax.rsqrt(d)[:, None]

    return pl.pallas_call(
        body,
        out_shape=jax.ShapeDtypeStruct((N_PAD, 1), jnp.float32),
    )(deg2)


def _tc_scale_mm(x, w, dis):
    def body(x_ref, w_ref, dis_ref, o_ref):
        h = jnp.dot(x_ref[...], w_ref[...], preferred_element_type=jnp.float32)
        o_ref[...] = h * dis_ref[...]

    return pl.pallas_call(
        body,
        grid=(_GRID,),
        in_specs=[
            pl.BlockSpec((_BLK, D), lambda i: (i, 0)),
            pl.BlockSpec((D, H), lambda i: (0, 0)),
            pl.BlockSpec((_BLK, 1), lambda i: (i, 0)),
        ],
        out_specs=pl.BlockSpec((_BLK, H), lambda i: (i, 0)),
        out_shape=jax.ShapeDtypeStruct((N, H), jnp.float32),
    )(x, w, dis)


def _tc_layer2(agg, hp, dis, w2, b1):
    def body(agg_ref, hp_ref, dis_ref, w_ref, b_ref, o_ref):
        s = agg_ref[0] + agg_ref[1] + hp_ref[...]
        a = jnp.maximum(s * dis_ref[...] + b_ref[...], 0.0)
        h = jnp.dot(a, w_ref[...], preferred_element_type=jnp.float32)
        o_ref[...] = h * dis_ref[...]

    return pl.pallas_call(
        body,
        grid=(_GRID,),
        in_specs=[
            pl.BlockSpec((NC, _BLK, H), lambda i: (0, i, 0)),
            pl.BlockSpec((_BLK, H), lambda i: (i, 0)),
            pl.BlockSpec((_BLK, 1), lambda i: (i, 0)),
            pl.BlockSpec((H, H), lambda i: (0, 0)),
            pl.BlockSpec((1, H), lambda i: (0, 0)),
        ],
        out_specs=pl.BlockSpec((_BLK, H), lambda i: (i, 0)),
        out_shape=jax.ShapeDtypeStruct((N, H), jnp.float32),
    )(agg, hp, dis, w2, b1)


def _tc_head(agg, hp, dis, b2, wfc, bfc):
    def body(agg_ref, hp_ref, dis_ref, b_ref, wfc_ref, bfc_ref, o_ref, acc):
        i = pl.program_id(0)
        s = agg_ref[0] + agg_ref[1] + hp_ref[...]
        h2 = jnp.maximum(s * dis_ref[...] + b_ref[...], 0.0)
        p = jnp.sum(h2, axis=0, keepdims=True)

        @pl.when(i == 0)
        def _():
            acc[...] = p

        @pl.when(i > 0)
        def _():
            acc[...] = acc[...] + p

        @pl.when(i == _GRID - 1)
        def _():
            g = acc[...] * (1.0 / N)
            o_ref[...] = (
                jnp.dot(g, wfc_ref[...], preferred_element_type=jnp.float32)
                + bfc_ref[...]
            )

    return pl.pallas_call(
        body,
        grid=(_GRID,),
        in_specs=[
            pl.BlockSpec((NC, _BLK, H), lambda i: (0, i, 0)),
            pl.BlockSpec((_BLK, H), lambda i: (i, 0)),
            pl.BlockSpec((_BLK, 1), lambda i: (i, 0)),
            pl.BlockSpec((1, H), lambda i: (0, 0)),
            pl.BlockSpec((H, O), lambda i: (0, 0)),
            pl.BlockSpec((1, O), lambda i: (0, 0)),
        ],
        out_specs=pl.BlockSpec((1, O), lambda i: (0, 0)),
        out_shape=jax.ShapeDtypeStruct((1, O), jnp.float32),
        scratch_shapes=[pltpu.VMEM((1, H), jnp.float32)],
    )(agg, hp, dis, b2, wfc, bfc)


def kernel(x, edge_index, W1, b1, W2, b2, Wfc, bfc):
    src = edge_index[0]
    dst = edge_index[1]
    pad = E_PAD - E
    src3 = jnp.concatenate([src, jnp.zeros((pad,), jnp.int32)]).reshape(
        NW, NCH, CH
    )
    dst3 = jnp.concatenate([dst, jnp.full((pad,), N, jnp.int32)]).reshape(
        NW, NCH, CH
    )

    deg2 = _sc_degree(dst3)
    dis = _tc_dis(deg2)
    hp1 = _tc_scale_mm(x, W1, dis)
    agg1 = _sc_agg(hp1, src3, dst3)
    hp2 = _tc_layer2(agg1, hp1, dis, W2, b1.reshape(1, H))
    agg2 = _sc_agg(hp2, src3, dst3)
    y = _tc_head(agg2, hp2, dis, b2.reshape(1, H), Wfc, bfc.reshape(1, O))
    return y.reshape(O)


# speedup vs baseline: 1.8420x; 1.8420x over previous
import functools

import jax
import jax.numpy as jnp
from jax import lax
from jax.experimental import pallas as pl
from jax.experimental.pallas import tpu as pltpu
from jax.experimental.pallas import tpu_sc as plsc

N = 10000
D = 128
H = 128
O = 64
E = 320000

NC = 2
NS = 16
NW = NC * NS
CH = 128
NCH = 79
EPW = NCH * CH
E_PAD = NW * EPW
N_PAD = 10240
RPT = N_PAD // NS
NR = 10112
RPA = NR // NS

_mesh = plsc.VectorSubcoreMesh(core_axis_name="c", subcore_axis_name="s")


def _fill_rows(ref, nrows, value):
    vec = jnp.full((16,), value, jnp.float32)

    def body(r, _):
        for cidx in range(CH // 16):
            ref[r, pl.ds(cidx * 16, 16)] = vec
        return 0

    lax.fori_loop(0, nrows, body, 0)


@functools.partial(
    pl.kernel,
    out_type=jax.ShapeDtypeStruct((NC, N_PAD), jnp.float32),
    mesh=_mesh,
    scratch_types=[
        pltpu.VMEM((NCH, CH), jnp.int32),
        pltpu.VMEM((CH,), jnp.float32),
        pltpu.VMEM((CH,), jnp.float32),
        pltpu.VMEM_SHARED((N_PAD,), jnp.float32),
    ],
)
def _sc_degree(dst_hbm, out_hbm, idx_v, ones_v, zero_v, deg_sh):
    cid = lax.axis_index("c")
    sid = lax.axis_index("s")
    wid = cid * NS + sid

    ones16 = jnp.ones((16,), jnp.float32)
    zeros16 = jnp.zeros((16,), jnp.float32)
    for i in range(CH // 16):
        ones_v[pl.ds(i * 16, 16)] = ones16
        zero_v[pl.ds(i * 16, 16)] = zeros16

    for k in range(RPT // CH):
        pltpu.sync_copy(zero_v, deg_sh.at[pl.ds(sid * RPT + k * CH, CH)])
    plsc.subcore_barrier()

    pltpu.sync_copy(dst_hbm.at[wid], idx_v)

    def body(j, _):
        pltpu.sync_copy(ones_v, deg_sh.at[idx_v.at[j]], add=True)
        return 0

    lax.fori_loop(0, NCH, body, 0)
    plsc.subcore_barrier()

    pltpu.sync_copy(
        deg_sh.at[pl.ds(sid * RPT, RPT)], out_hbm.at[cid, pl.ds(sid * RPT, RPT)]
    )


@functools.partial(
    pl.kernel,
    out_type=jax.ShapeDtypeStruct((NC, NR, H), jnp.float32),
    mesh=_mesh,
    scratch_types=[
        pltpu.VMEM((NCH, CH), jnp.int32),
        pltpu.VMEM((NCH, CH), jnp.int32),
        pltpu.VMEM((CH, H), jnp.float32),
        pltpu.VMEM_SHARED((NR, H), jnp.float32),
        pltpu.SemaphoreType.DMA,
    ],
)
def _sc_agg(hp_hbm, src_hbm, dst_hbm, out_hbm, src_v, dst_v, ra, agg_sh,
            sem_a):
    cid = lax.axis_index("c")
    sid = lax.axis_index("s")
    wid = cid * NS + sid

    _fill_rows(ra, CH, 0.0)
    base = sid * RPA
    for k in range(RPA // CH):
        pltpu.sync_copy(ra, agg_sh.at[pl.ds(base + k * CH, CH)])
    if RPA % CH:
        pltpu.sync_copy(ra, agg_sh.at[pl.ds(base + RPA - CH, CH)])
    plsc.subcore_barrier()

    pltpu.sync_copy(src_hbm.at[wid], src_v)
    pltpu.sync_copy(dst_hbm.at[wid], dst_v)

    def body(j, _):
        pltpu.async_copy(hp_hbm.at[src_v.at[j]], ra, sem_a)
        pltpu.make_async_copy(hp_hbm.at[src_v.at[j]], ra, sem_a).wait()
        pltpu.sync_copy(ra, agg_sh.at[dst_v.at[j]], add=True)
        return 0

    lax.fori_loop(0, NCH, body, 0)
    plsc.subcore_barrier()

    pltpu.sync_copy(
        agg_sh.at[pl.ds(sid * RPA, RPA)],
        out_hbm.at[cid, pl.ds(sid * RPA, RPA), :],
    )


_BLK = 2000
_GRID = N // _BLK


def _tc_dis(deg2):
    def body(deg_ref, o_ref):
        d = deg_ref[0, :] + deg_ref[1, :] + 1.0
        o_ref[...] = lax.rsqrt(d)[:, None]

    return pl.pallas_call(
        body,
        out_shape=jax.ShapeDtypeStruct((N_PAD, 1), jnp.float32),
    )(deg2)


def _tc_scale_mm(x, w, dis):
    def body(x_ref, w_ref, dis_ref, o_ref):
        h = jnp.dot(x_ref[...], w_ref[...], preferred_element_type=jnp.float32)
        o_ref[...] = h * dis_ref[...]

    return pl.pallas_call(
        body,
        grid=(_GRID,),
        in_specs=[
            pl.BlockSpec((_BLK, D), lambda i: (i, 0)),
            pl.BlockSpec((D, H), lambda i: (0, 0)),
            pl.BlockSpec((_BLK, 1), lambda i: (i, 0)),
        ],
        out_specs=pl.BlockSpec((_BLK, H), lambda i: (i, 0)),
        out_shape=jax.ShapeDtypeStruct((N, H), jnp.float32),
    )(x, w, dis)


def _tc_layer2(agg, hp, dis, w2, b1):
    def body(agg_ref, hp_ref, dis_ref, w_ref, b_ref, o_ref):
        s = agg_ref[0] + agg_ref[1] + hp_ref[...]
        a = jnp.maximum(s * dis_ref[...] + b_ref[...], 0.0)
        h = jnp.dot(a, w_ref[...], preferred_element_type=jnp.float32)
        o_ref[...] = h * dis_ref[...]

    return pl.pallas_call(
        body,
        grid=(_GRID,),
        in_specs=[
            pl.BlockSpec((NC, _BLK, H), lambda i: (0, i, 0)),
            pl.BlockSpec((_BLK, H), lambda i: (i, 0)),
            pl.BlockSpec((_BLK, 1), lambda i: (i, 0)),
            pl.BlockSpec((H, H), lambda i: (0, 0)),
            pl.BlockSpec((1, H), lambda i: (0, 0)),
        ],
        out_specs=pl.BlockSpec((_BLK, H), lambda i: (i, 0)),
        out_shape=jax.ShapeDtypeStruct((N, H), jnp.float32),
    )(agg, hp, dis, w2, b1)


def _tc_head(agg, hp, dis, b2, wfc, bfc):
    def body(agg_ref, hp_ref, dis_ref, b_ref, wfc_ref, bfc_ref, o_ref, acc):
        i = pl.program_id(0)
        s = agg_ref[0] + agg_ref[1] + hp_ref[...]
        h2 = jnp.maximum(s * dis_ref[...] + b_ref[...], 0.0)
        p = jnp.sum(h2, axis=0, keepdims=True)

        @pl.when(i == 0)
        def _():
            acc[...] = p

        @pl.when(i > 0)
        def _():
            acc[...] = acc[...] + p

        @pl.when(i == _GRID - 1)
        def _():
            g = acc[...] * (1.0 / N)
            o_ref[...] = (
                jnp.dot(g, wfc_ref[...], preferred_element_type=jnp.float32)
                + bfc_ref[...]
            )

    return pl.pallas_call(
        body,
        grid=(_GRID,),
        in_specs=[
            pl.BlockSpec((NC, _BLK, H), lambda i: (0, i, 0)),
            pl.BlockSpec((_BLK, H), lambda i: (i, 0)),
            pl.BlockSpec((_BLK, 1), lambda i: (i, 0)),
            pl.BlockSpec((1, H), lambda i: (0, 0)),
            pl.BlockSpec((H, O), lambda i: (0, 0)),
            pl.BlockSpec((1, O), lambda i: (0, 0)),
        ],
        out_specs=pl.BlockSpec((1, O), lambda i: (0, 0)),
        out_shape=jax.ShapeDtypeStruct((1, O), jnp.float32),
        scratch_shapes=[pltpu.VMEM((1, H), jnp.float32)],
    )(agg, hp, dis, b2, wfc, bfc)


def kernel(x, edge_index, W1, b1, W2, b2, Wfc, bfc):
    src = edge_index[0]
    dst = edge_index[1]
    pad = E_PAD - E
    ar = jnp.arange(pad, dtype=jnp.int32)
    src3 = jnp.concatenate([src, ar % N]).reshape(NW, NCH, CH)
    dst3 = jnp.concatenate([dst, N + (ar % (NR - N))]).reshape(NW, NCH, CH)

    deg2 = _sc_degree(dst3)
    dis = _tc_dis(deg2)
    hp1 = _tc_scale_mm(x, W1, dis)
    agg1 = _sc_agg(hp1, src3, dst3)
    hp2 = _tc_layer2(agg1, hp1, dis, W2, b1.reshape(1, H))
    agg2 = _sc_agg(hp2, src3, dst3)
    y = _tc_head(agg2, hp2, dis, b2.reshape(1, H), Wfc, bfc.reshape(1, O))
    return y.reshape(O)
